# Initial kernel scaffold; baseline (speedup 1.0000x reference)
import jax
import jax.numpy as jnp
from jax.experimental import pallas as pl


def kernel(x, edge_index, W1, b1, W2, b2, gamma, beta, Wc, bc):
    raise NotImplementedError("write your pallas kernel here")


# speedup vs baseline: 4.5518x; 4.5518x over previous
import functools

import jax
import jax.numpy as jnp
from jax import lax
from jax.experimental import pallas as pl
from jax.experimental.pallas import tpu as pltpu
from jax.experimental.pallas import tpu_sc as plsc

N = 10000
E = 320000
F_IN = 128
H = 64

NC = 2
NS = 16
NW = NC * NS
CHUNK = 128
NCH = -(-E // (NW * CHUNK))
EPT = NCH * CHUNK
EPAD = EPT * NW
ZR = 632
NPAD = NS * ZR


def _epilogue_body(x_ref, p_ref, w1_ref, b1_ref, w2_ref, b2_ref, g_ref, be_ref,
                   wc_ref, bc_ref, o_ref):
    h = x_ref[...] + p_ref[0, :N, :] + p_ref[1, :N, :]
    h = jnp.maximum(
        jnp.dot(h, w1_ref[...], preferred_element_type=jnp.float32) + b1_ref[...],
        0.0)
    h = jnp.dot(h, w2_ref[...], preferred_element_type=jnp.float32) + b2_ref[...]
    mean = jnp.mean(h, axis=0, keepdims=True)
    var = jnp.mean((h - mean) ** 2, axis=0, keepdims=True)
    h = (h - mean) * lax.rsqrt(var + 1e-5) * g_ref[...] + be_ref[...]
    h = jnp.maximum(h, 0.0)
    o_ref[...] = jnp.dot(h, wc_ref[...], preferred_element_type=jnp.float32) + bc_ref[...]


def _sc_scatter_body(x_hbm, srcs_hbm, dsts_hbm, zeros_hbm, out_hbm,
                     src_v, dst_v, rows, acc, sem):
    cid = lax.axis_index("c")
    sid = lax.axis_index("s")
    wid = cid * NS + sid
    pltpu.sync_copy(zeros_hbm.at[pl.ds(sid * ZR, ZR)], acc.at[pl.ds(sid * ZR, ZR)])
    pltpu.sync_copy(srcs_hbm.at[wid], src_v)
    pltpu.sync_copy(dsts_hbm.at[wid], dst_v)
    plsc.subcore_barrier()

    @pl.loop(0, NCH)
    def _chunk(j):
        pltpu.async_copy(x_hbm.at[src_v.at[j]], rows, sem).wait()
        pltpu.sync_copy(rows, acc.at[dst_v.at[j]], add=True)

    plsc.subcore_barrier()
    pltpu.sync_copy(acc.at[pl.ds(sid * ZR, ZR)],
                    out_hbm.at[cid, pl.ds(sid * ZR, ZR)])


_sc_scatter = functools.partial(
    pl.kernel,
    out_type=jax.ShapeDtypeStruct((NC, NPAD, F_IN), jnp.float32),
    mesh=plsc.VectorSubcoreMesh(core_axis_name="c", subcore_axis_name="s"),
    scratch_types=[
        pltpu.VMEM((NCH, CHUNK), jnp.int32),
        pltpu.VMEM((NCH, CHUNK), jnp.int32),
        pltpu.VMEM((CHUNK, F_IN), jnp.float32),
        pltpu.VMEM_SHARED((NPAD, F_IN), jnp.float32),
        pltpu.SemaphoreType.DMA,
    ],
)(_sc_scatter_body)


def kernel(x, edge_index, W1, b1, W2, b2, gamma, beta, Wc, bc):
    src = edge_index[0]
    dst = edge_index[1]
    pad = EPAD - E
    srcs = jnp.concatenate([src, jnp.zeros((pad,), jnp.int32)]).reshape(NW, NCH, CHUNK)
    dsts = jnp.concatenate([dst, jnp.full((pad,), N, jnp.int32)]).reshape(NW, NCH, CHUNK)
    zeros = jnp.zeros((NPAD, F_IN), jnp.float32)

    parts = _sc_scatter(x, srcs, dsts, zeros)

    out = pl.pallas_call(
        _epilogue_body,
        out_shape=jax.ShapeDtypeStruct((N, 2), jnp.float32),
    )(x, parts, W1, b1.reshape(1, H), W2, b2.reshape(1, H),
      gamma.reshape(1, H), beta.reshape(1, H), Wc, bc.reshape(1, 2))
    return out
